# baseline (speedup 1.0000x reference)
import functools

import jax
import jax.numpy as jnp
from jax import lax
from jax.experimental import pallas as pl
from jax.experimental.pallas import tpu as pltpu
from jax.experimental.pallas import tpu_sc as plsc

_N = 100000
_R = 100352
_EP = 3309568
_K = 1024
_CH_EX = 101
_CH_NUM = 202
_RPW = _R // 16


def _ex_body(src1, dst1, U, V, X12,
             exw, xs12,
             idx_s, idx_d, u_b, v_b, x_b, sem):
    c = lax.axis_index("c")
    s = lax.axis_index("s")
    w = s * 2 + c

    def chunk(g, carry):
        e0 = (w * _CH_EX + g) * _K
        pltpu.sync_copy(src1.at[pl.ds(e0, _K)], idx_s)
        pltpu.sync_copy(dst1.at[pl.ds(e0, _K)], idx_d)
        cps = [pltpu.async_copy(U.at[idx_s], u_b, sem),
               pltpu.async_copy(V.at[idx_d], v_b, sem),
               pltpu.async_copy(X12.at[idx_s], x_b, sem)]
        for cp in cps:
            cp.wait()

        def vec(t, carry2):
            for i in range(4):
                r = t * 4 + i
                u16 = u_b[r, pl.ds(0, 16)]
                v16 = v_b[r, pl.ds(0, 16)]
                e = u16 + v16
                e = jnp.where(e >= 0.0, e, 0.2 * e)
                u_b[r, pl.ds(0, 16)] = jnp.exp(e)
            return carry2

        lax.fori_loop(0, _K // 4, vec, 0)
        pltpu.sync_copy(u_b, exw.at[pl.ds(e0, _K), :])
        pltpu.sync_copy(x_b, xs12.at[pl.ds(e0, _K), :])
        return carry

    lax.fori_loop(0, _CH_EX, chunk, 0)


_KN = 512
_CH_KN = _EP // (16 * _KN)


def _num_body(dst1, exw, XS, Z16,
              o0A, o0B, o1A, o1B, o2A, o2B, o3A, o3B,
              idx_d, ex_b, x_b, num_sh, sem):
    c = lax.axis_index("c")
    s = lax.axis_index("s")
    outs = ((o0A, o0B), (o1A, o1B), (o2A, o2B), (o3A, o3B))

    for lidx in range(4):
        off = 0 if lidx < 2 else 24
        outA, outB = outs[lidx]
        pltpu.sync_copy(Z16.at[pl.ds(s * _RPW, _RPW), :],
                        num_sh.at[pl.ds(s * _RPW, _RPW), :])
        plsc.subcore_barrier()

        def chunk(g, carry, lidx=lidx, off=off):
            e0 = (s * _CH_KN + g) * _KN
            pltpu.sync_copy(dst1.at[pl.ds(e0, _KN)], idx_d)
            pltpu.sync_copy(exw.at[pl.ds(e0, _KN), :], ex_b)

            @pl.when(c == 0)
            def _():
                pltpu.sync_copy(XS.at[pl.ds(e0, _KN), pl.ds(off, 16)], x_b)

            @pl.when(c == 1)
            def _():
                pltpu.sync_copy(XS.at[pl.ds(e0, _KN), pl.ds(off + 8, 16)],
                                x_b)

            def grp(t, carry2):
                for i in range(4):
                    r = t * 4 + i
                    exrow = ex_b[r, pl.ds(0, 16)]
                    w16 = jnp.broadcast_to(
                        lax.slice(exrow, (lidx,), (lidx + 1,)), (16,))
                    x_b[r, pl.ds(0, 16)] = x_b[r, pl.ds(0, 16)] * w16
                return carry2

            lax.fori_loop(0, _KN // 4, grp, 0)
            pltpu.sync_copy(x_b, num_sh.at[idx_d], add=True)
            return carry

        lax.fori_loop(0, _CH_KN, chunk, 0)
        plsc.subcore_barrier()

        @pl.when(c == 0)
        def _(outA=outA):
            pltpu.sync_copy(num_sh.at[pl.ds(s * _RPW, _RPW), :],
                            outA.at[pl.ds(s * _RPW, _RPW), :])

        @pl.when(c == 1)
        def _(outB=outB):
            pltpu.sync_copy(num_sh.at[pl.ds(s * _RPW, _RPW), :],
                            outB.at[pl.ds(s * _RPW, _RPW), :])


def _mesh():
    return plsc.VectorSubcoreMesh(core_axis_name="c", subcore_axis_name="s")


_SC_PARAMS = pltpu.CompilerParams(use_tc_tiling_on_sc=False)


def _ex_call(src1, dst1, U, V, X12):
    f32 = jnp.float32
    return pl.kernel(
        _ex_body,
        out_type=[jax.ShapeDtypeStruct((_EP, 16), f32),
                  jax.ShapeDtypeStruct((_EP, 48), f32)],
        mesh=_mesh(),
        compiler_params=_SC_PARAMS,
        scratch_types=[
            pltpu.VMEM((_K,), jnp.int32),
            pltpu.VMEM((_K,), jnp.int32),
            pltpu.VMEM((_K, 16), f32),
            pltpu.VMEM((_K, 16), f32),
            pltpu.VMEM((_K, 48), f32),
            pltpu.SemaphoreType.DMA,
        ],
    )(src1, dst1, U, V, X12)


def _num_call(dst1, exw, XS, Z16):
    f32 = jnp.float32
    res = pl.kernel(
        _num_body,
        out_type=[jax.ShapeDtypeStruct((_R, 16), f32)] * 8,
        mesh=_mesh(),
        compiler_params=_SC_PARAMS,
        scratch_types=[
            pltpu.VMEM((_KN,), jnp.int32),
            pltpu.VMEM((_KN, 16), f32),
            pltpu.VMEM((_KN, 16), f32),
            pltpu.VMEM_SHARED((_R, 16), f32),
            pltpu.SemaphoreType.DMA,
        ],
    )(dst1, exw, XS, Z16)
    nums, dens = [], []
    for l in range(4):
        outA, outB = res[2 * l], res[2 * l + 1]
        nums.append(jnp.concatenate([outA, outB[:, 8:12]], axis=1))
        dens.append(outB[:, 12])
    return nums, dens


_BLK = 2000


def _final_body(num1, num2, num3, num4, dens, x2, W1, W2, W3, W4, bias,
                x1n_o, x2n_o, ld_o):
    inv = 1.0 / (dens[...] + 1e-16)
    s1 = jnp.dot(num1[...], W1[...], preferred_element_type=jnp.float32) * inv[:, 0:1] + bias[0, :][None, :]
    t1 = jnp.dot(num2[...], W2[...], preferred_element_type=jnp.float32) * inv[:, 1:2] + bias[1, :][None, :]
    s2 = jnp.dot(num3[...], W3[...], preferred_element_type=jnp.float32) * inv[:, 2:3] + bias[2, :][None, :]
    t2 = jnp.dot(num4[...], W4[...], preferred_element_type=jnp.float32) * inv[:, 3:4] + bias[3, :][None, :]
    x1n = x2[...] * jnp.exp(s1) + t1
    x2n = x1n * jnp.exp(s2) + t2
    x1n_o[...] = x1n
    x2n_o[...] = x2n
    ld_o[...] = jnp.sum(s1 + s2, axis=1, keepdims=True)


def _final_stage(num1, num2, num3, num4, dens, x2, W1, W2, W3, W4, bias):
    n = num1.shape[0]
    grid = (n // _BLK,)
    row = lambda i: (i, 0)
    full = lambda i: (0, 0)
    x1n, x2n, ld = pl.pallas_call(
        _final_body,
        grid=grid,
        in_specs=[
            pl.BlockSpec((_BLK, 20), row),
            pl.BlockSpec((_BLK, 20), row),
            pl.BlockSpec((_BLK, 20), row),
            pl.BlockSpec((_BLK, 20), row),
            pl.BlockSpec((_BLK, 4), row),
            pl.BlockSpec((_BLK, 20), row),
            pl.BlockSpec((20, 20), full),
            pl.BlockSpec((20, 20), full),
            pl.BlockSpec((20, 20), full),
            pl.BlockSpec((20, 20), full),
            pl.BlockSpec((4, 20), full),
        ],
        out_specs=[
            pl.BlockSpec((_BLK, 20), row),
            pl.BlockSpec((_BLK, 20), row),
            pl.BlockSpec((_BLK, 1), row),
        ],
        out_shape=[
            jax.ShapeDtypeStruct((n, 20), jnp.float32),
            jax.ShapeDtypeStruct((n, 20), jnp.float32),
            jax.ShapeDtypeStruct((n, 1), jnp.float32),
        ],
    )(num1, num2, num3, num4, dens, x2, W1, W2, W3, W4, bias)
    return x1n, x2n, ld[:, 0]


def kernel(x, edge_index, F1_W, F1_as, F1_ad, F1_b, F2_W, F2_as, F2_ad, F2_b,
           G1_W, G1_as, G1_ad, G1_b, G2_W, G2_as, G2_ad, G2_b):
    n = x.shape[0]
    loops = jnp.arange(n, dtype=edge_index.dtype)
    src = jnp.concatenate([edge_index[0], loops])
    dst = jnp.concatenate([edge_index[1], loops])
    ea = src.shape[0]
    pad = jnp.full((_EP - ea,), _N, dtype=jnp.int32)
    src1 = jnp.concatenate([src, pad])
    dst1 = jnp.concatenate([dst, pad])
    x1, x2 = x[:, :20], x[:, 20:]

    U4 = jnp.stack([x1 @ (F1_W @ F1_as), x1 @ (F2_W @ F2_as),
                    x2 @ (G1_W @ G1_as), x2 @ (G2_W @ G2_as)], axis=1)
    V4 = jnp.stack([x1 @ (F1_W @ F1_ad), x1 @ (F2_W @ F2_ad),
                    x2 @ (G1_W @ G1_ad), x2 @ (G2_W @ G2_ad)], axis=1)
    U = jnp.pad(U4, ((0, _R - _N), (0, 12)))
    V = jnp.pad(V4, ((0, _R - _N), (0, 12)))
    ones = jnp.ones((_R, 1), jnp.float32)
    zpad = jnp.zeros((_R, 3), jnp.float32)
    rp = ((0, _R - _N), (0, 0))
    X12 = jnp.concatenate([jnp.pad(x1, rp), ones, zpad,
                           jnp.pad(x2, rp), ones, zpad], axis=1)
    Z16 = jnp.zeros((_R, 16), jnp.float32)

    exw, xs12 = _ex_call(src1, dst1, U, V, X12)
    (num1, num2, num3, num4), dlist = _num_call(dst1, exw, xs12, Z16)
    dens = jnp.stack(dlist, axis=1)[:n]

    bias = jnp.stack([F1_b, F2_b, G1_b, G2_b], axis=0)
    x1n, x2n, ld = _final_stage(num1[:n], num2[:n], num3[:n], num4[:n],
                                dens, x2, F1_W, F2_W, G1_W, G2_W, bias)
    return (x1n, x2n, ld)


# speedup vs baseline: 1.0502x; 1.0502x over previous
import functools

import jax
import jax.numpy as jnp
from jax import lax
from jax.experimental import pallas as pl
from jax.experimental.pallas import tpu as pltpu
from jax.experimental.pallas import tpu_sc as plsc

_N = 100000
_R = 100352
_EP = 3309568
_K = 1024
_CH_EX = 101
_CH_NUM = 202
_RPW = _R // 16


def _ex_body(src1, dst1, U, V, X12,
             exw, xs12,
             idx_s, idx_d, u_b, v_b, x_b, sem):
    c = lax.axis_index("c")
    s = lax.axis_index("s")
    w = s * 2 + c

    def chunk(g, carry):
        e0 = (w * _CH_EX + g) * _K
        pltpu.sync_copy(src1.at[pl.ds(e0, _K)], idx_s)
        pltpu.sync_copy(dst1.at[pl.ds(e0, _K)], idx_d)
        cps = [pltpu.async_copy(U.at[idx_s], u_b, sem),
               pltpu.async_copy(V.at[idx_d], v_b, sem),
               pltpu.async_copy(X12.at[idx_s], x_b, sem)]
        for cp in cps:
            cp.wait()

        def vec(t, carry2):
            for i in range(4):
                r = t * 4 + i
                u16 = u_b[r, pl.ds(0, 16)]
                v16 = v_b[r, pl.ds(0, 16)]
                e = u16 + v16
                e = jnp.where(e >= 0.0, e, 0.2 * e)
                u_b[r, pl.ds(0, 16)] = jnp.exp(e)
            return carry2

        lax.fori_loop(0, _K // 4, vec, 0)
        pltpu.sync_copy(u_b, exw.at[pl.ds(e0, _K), :])
        pltpu.sync_copy(x_b, xs12.at[pl.ds(e0, _K), :])
        return carry

    lax.fori_loop(0, _CH_EX, chunk, 0)


_KN = 1024
_CH_KN = _EP // (16 * _KN)


def _num_body(dst1, ex4, XS, Z16,
              o0A, o0B, o1A, o1B, o2A, o2B, o3A, o3B,
              idx_d, ex_b, x_b, num_sh, sem):
    c = lax.axis_index("c")
    s = lax.axis_index("s")
    outs = ((o0A, o0B), (o1A, o1B), (o2A, o2B), (o3A, o3B))

    for lidx in range(4):
        off = 0 if lidx < 2 else 24
        outA, outB = outs[lidx]
        pltpu.sync_copy(Z16.at[pl.ds(s * _RPW, _RPW), :],
                        num_sh.at[pl.ds(s * _RPW, _RPW), :])
        plsc.subcore_barrier()

        def chunk(g, carry, lidx=lidx, off=off):
            e0 = (s * _CH_KN + g) * _KN
            pltpu.sync_copy(dst1.at[pl.ds(e0, _KN)], idx_d)
            pltpu.sync_copy(ex4.at[lidx, pl.ds(e0, _KN)], ex_b)

            @pl.when(c == 0)
            def _():
                pltpu.sync_copy(XS.at[pl.ds(e0, _KN), pl.ds(off, 16)], x_b)

            @pl.when(c == 1)
            def _():
                pltpu.sync_copy(XS.at[pl.ds(e0, _KN), pl.ds(off + 8, 16)],
                                x_b)

            def grp(t, carry2):
                ex16 = ex_b[pl.ds(t * 16, 16)]
                for i in range(16):
                    w16 = jnp.broadcast_to(
                        lax.slice(ex16, (i,), (i + 1,)), (16,))
                    r = t * 16 + i
                    x_b[r, pl.ds(0, 16)] = x_b[r, pl.ds(0, 16)] * w16
                return carry2

            lax.fori_loop(0, _KN // 16, grp, 0)
            pltpu.sync_copy(x_b, num_sh.at[idx_d], add=True)
            return carry

        lax.fori_loop(0, _CH_KN, chunk, 0)
        plsc.subcore_barrier()

        @pl.when(c == 0)
        def _(outA=outA):
            pltpu.sync_copy(num_sh.at[pl.ds(s * _RPW, _RPW), :],
                            outA.at[pl.ds(s * _RPW, _RPW), :])

        @pl.when(c == 1)
        def _(outB=outB):
            pltpu.sync_copy(num_sh.at[pl.ds(s * _RPW, _RPW), :],
                            outB.at[pl.ds(s * _RPW, _RPW), :])


def _mesh():
    return plsc.VectorSubcoreMesh(core_axis_name="c", subcore_axis_name="s")


_SC_PARAMS = pltpu.CompilerParams(use_tc_tiling_on_sc=False)


def _ex_call(src1, dst1, U, V, X12):
    f32 = jnp.float32
    return pl.kernel(
        _ex_body,
        out_type=[jax.ShapeDtypeStruct((_EP, 16), f32),
                  jax.ShapeDtypeStruct((_EP, 48), f32)],
        mesh=_mesh(),
        compiler_params=_SC_PARAMS,
        scratch_types=[
            pltpu.VMEM((_K,), jnp.int32),
            pltpu.VMEM((_K,), jnp.int32),
            pltpu.VMEM((_K, 16), f32),
            pltpu.VMEM((_K, 16), f32),
            pltpu.VMEM((_K, 48), f32),
            pltpu.SemaphoreType.DMA,
        ],
    )(src1, dst1, U, V, X12)


def _num_call(dst1, ex4, XS, Z16):
    f32 = jnp.float32
    res = pl.kernel(
        _num_body,
        out_type=[jax.ShapeDtypeStruct((_R, 16), f32)] * 8,
        mesh=_mesh(),
        compiler_params=_SC_PARAMS,
        scratch_types=[
            pltpu.VMEM((_KN,), jnp.int32),
            pltpu.VMEM((_KN,), f32),
            pltpu.VMEM((_KN, 16), f32),
            pltpu.VMEM_SHARED((_R, 16), f32),
            pltpu.SemaphoreType.DMA,
        ],
    )(dst1, ex4, XS, Z16)
    nums, dens = [], []
    for l in range(4):
        outA, outB = res[2 * l], res[2 * l + 1]
        nums.append(jnp.concatenate([outA, outB[:, 8:12]], axis=1))
        dens.append(outB[:, 12])
    return nums, dens


_SB = 8192


def _split_body(exw_blk, out_blk):
    out_blk[...] = jnp.transpose(exw_blk[:, 0:4])


def _split_stage(exw):
    return pl.pallas_call(
        _split_body,
        grid=(_EP // _SB,),
        in_specs=[pl.BlockSpec((_SB, 16), lambda i: (i, 0))],
        out_specs=pl.BlockSpec((4, _SB), lambda i: (0, i)),
        out_shape=jax.ShapeDtypeStruct((4, _EP), jnp.float32),
    )(exw)


_BLK = 2000


def _final_body(num1, num2, num3, num4, dens, x2, W1, W2, W3, W4, bias,
                x1n_o, x2n_o, ld_o):
    inv = 1.0 / (dens[...] + 1e-16)
    s1 = jnp.dot(num1[...], W1[...], preferred_element_type=jnp.float32) * inv[:, 0:1] + bias[0, :][None, :]
    t1 = jnp.dot(num2[...], W2[...], preferred_element_type=jnp.float32) * inv[:, 1:2] + bias[1, :][None, :]
    s2 = jnp.dot(num3[...], W3[...], preferred_element_type=jnp.float32) * inv[:, 2:3] + bias[2, :][None, :]
    t2 = jnp.dot(num4[...], W4[...], preferred_element_type=jnp.float32) * inv[:, 3:4] + bias[3, :][None, :]
    x1n = x2[...] * jnp.exp(s1) + t1
    x2n = x1n * jnp.exp(s2) + t2
    x1n_o[...] = x1n
    x2n_o[...] = x2n
    ld_o[...] = jnp.sum(s1 + s2, axis=1, keepdims=True)


def _final_stage(num1, num2, num3, num4, dens, x2, W1, W2, W3, W4, bias):
    n = num1.shape[0]
    grid = (n // _BLK,)
    row = lambda i: (i, 0)
    full = lambda i: (0, 0)
    x1n, x2n, ld = pl.pallas_call(
        _final_body,
        grid=grid,
        in_specs=[
            pl.BlockSpec((_BLK, 20), row),
            pl.BlockSpec((_BLK, 20), row),
            pl.BlockSpec((_BLK, 20), row),
            pl.BlockSpec((_BLK, 20), row),
            pl.BlockSpec((_BLK, 4), row),
            pl.BlockSpec((_BLK, 20), row),
            pl.BlockSpec((20, 20), full),
            pl.BlockSpec((20, 20), full),
            pl.BlockSpec((20, 20), full),
            pl.BlockSpec((20, 20), full),
            pl.BlockSpec((4, 20), full),
        ],
        out_specs=[
            pl.BlockSpec((_BLK, 20), row),
            pl.BlockSpec((_BLK, 20), row),
            pl.BlockSpec((_BLK, 1), row),
        ],
        out_shape=[
            jax.ShapeDtypeStruct((n, 20), jnp.float32),
            jax.ShapeDtypeStruct((n, 20), jnp.float32),
            jax.ShapeDtypeStruct((n, 1), jnp.float32),
        ],
    )(num1, num2, num3, num4, dens, x2, W1, W2, W3, W4, bias)
    return x1n, x2n, ld[:, 0]


def kernel(x, edge_index, F1_W, F1_as, F1_ad, F1_b, F2_W, F2_as, F2_ad, F2_b,
           G1_W, G1_as, G1_ad, G1_b, G2_W, G2_as, G2_ad, G2_b):
    n = x.shape[0]
    loops = jnp.arange(n, dtype=edge_index.dtype)
    src = jnp.concatenate([edge_index[0], loops])
    dst = jnp.concatenate([edge_index[1], loops])
    ea = src.shape[0]
    pad = jnp.full((_EP - ea,), _N, dtype=jnp.int32)
    src1 = jnp.concatenate([src, pad])
    dst1 = jnp.concatenate([dst, pad])
    x1, x2 = x[:, :20], x[:, 20:]

    U4 = jnp.stack([x1 @ (F1_W @ F1_as), x1 @ (F2_W @ F2_as),
                    x2 @ (G1_W @ G1_as), x2 @ (G2_W @ G2_as)], axis=1)
    V4 = jnp.stack([x1 @ (F1_W @ F1_ad), x1 @ (F2_W @ F2_ad),
                    x2 @ (G1_W @ G1_ad), x2 @ (G2_W @ G2_ad)], axis=1)
    U = jnp.pad(U4, ((0, _R - _N), (0, 12)))
    V = jnp.pad(V4, ((0, _R - _N), (0, 12)))
    ones = jnp.ones((_R, 1), jnp.float32)
    zpad = jnp.zeros((_R, 3), jnp.float32)
    rp = ((0, _R - _N), (0, 0))
    X12 = jnp.concatenate([jnp.pad(x1, rp), ones, zpad,
                           jnp.pad(x2, rp), ones, zpad], axis=1)
    Z16 = jnp.zeros((_R, 16), jnp.float32)

    exw, xs12 = _ex_call(src1, dst1, U, V, X12)
    ex4 = _split_stage(exw)
    (num1, num2, num3, num4), dlist = _num_call(dst1, ex4, xs12, Z16)
    dens = jnp.stack(dlist, axis=1)[:n]

    bias = jnp.stack([F1_b, F2_b, G1_b, G2_b], axis=0)
    x1n, x2n, ld = _final_stage(num1[:n], num2[:n], num3[:n], num4[:n],
                                dens, x2, F1_W, F2_W, G1_W, G2_W, bias)
    return (x1n, x2n, ld)
